# baseline (speedup 1.0000x reference)
import functools

import jax
import jax.numpy as jnp
from jax import lax
from jax.experimental import pallas as pl
from jax.experimental.pallas import tpu as pltpu
from jax.experimental.pallas import tpu_sc as plsc

K = 8192
D = 256
B = 4096

BB = 256
BK = 2048
NB = B // BB
NK = K // BK
NS = NK * NB

_NC = 2
_NS = 16
_NW = _NC * _NS
_BPW = B // _NW


def _mm_argmax(z_ref, e_ref, out_ref, idx_ref, buf_ref, max_ref, arg_ref):
    s = pl.program_id(1)

    @pl.when(s > 0)
    def _():
        blk = buf_ref[...]
        sp = s - 1
        bkp = sp // NB
        bbp = lax.rem(sp, NB)
        m = blk[:, :1]
        cols = lax.broadcasted_iota(jnp.int32, (BB, BK), 1)
        inblk = cols[:, :1]
        gidx = bkp * BK + inblk
        rows = pl.ds(bbp * BB, BB)

        @pl.when(bkp == 0)
        def _():
            max_ref[rows, :] = m
            arg_ref[rows, :] = gidx

        @pl.when(bkp > 0)
        def _():
            cur = max_ref[rows, :]
            better = m > cur
            max_ref[rows, :] = jnp.where(better, m, cur)
            arg_ref[rows, :] = jnp.where(better, gidx, arg_ref[rows, :])

        @pl.when(bkp == NK - 1)
        def _():
            idx_ref[0] = arg_ref[rows, :]

    @pl.when(s < NS)
    def _():
        blk = lax.dot_general(z_ref[...], e_ref[...], (((1,), (1,)), ((), ())),
                              preferred_element_type=jnp.float32)
        out_ref[...] = blk
        buf_ref[...] = blk


def _logits_and_idx(z_e_x, emb_flat):
    def zmap(c, s):
        sp = jnp.minimum(s, NS - 1)
        return (lax.rem(sp, NB), c)

    def emap(c, s):
        sp = jnp.minimum(s, NS - 1)
        return (c * NK + sp // NB, 0)

    def omap(c, s):
        sp = jnp.minimum(s, NS - 1)
        return (lax.rem(sp, NB), c * NK + sp // NB)

    def imap(c, s):
        sp = jnp.maximum(s, 1) - 1
        return (c, lax.rem(sp, NB), 0)

    return pl.pallas_call(
        _mm_argmax,
        grid=(2, NS + 1),
        in_specs=[
            pl.BlockSpec((BB, D), zmap),
            pl.BlockSpec((BK, D), emap),
        ],
        out_specs=[
            pl.BlockSpec((BB, BK), omap),
            pl.BlockSpec((1, BB, 1), imap),
        ],
        out_shape=[
            jax.ShapeDtypeStruct((B, 2 * K), jnp.float32),
            jax.ShapeDtypeStruct((2, B, 1), jnp.int32),
        ],
        scratch_shapes=[
            pltpu.VMEM((BB, BK), jnp.float32),
            pltpu.VMEM((B, 1), jnp.float32),
            pltpu.VMEM((B, 1), jnp.int32),
        ],
        compiler_params=pltpu.CompilerParams(
            dimension_semantics=("arbitrary", "arbitrary")),
    )(z_e_x, emb_flat)


def _sc_gather(emb_a, emb_v, idx_flat):
    mesh = plsc.VectorSubcoreMesh(core_axis_name="c", subcore_axis_name="s")

    @functools.partial(
        pl.kernel, mesh=mesh,
        out_type=jax.ShapeDtypeStruct((2 * B, D), jnp.float32),
        scratch_types=[
            pltpu.VMEM((_BPW,), jnp.int32),
            pltpu.VMEM((_BPW, D), jnp.float32),
            pltpu.SemaphoreType.DMA,
        ],
    )
    def gather(ea_hbm, ev_hbm, idx_hbm, out_hbm, idx_v, rows_v, sem):
        wid = lax.axis_index("s") * _NC + lax.axis_index("c")
        for t, tab in ((0, ea_hbm), (1, ev_hbm)):
            base = t * B + wid * _BPW
            pltpu.sync_copy(idx_hbm.at[pl.ds(base, _BPW)], idx_v)
            pltpu.async_copy(tab.at[idx_v], rows_v, sem).wait()
            pltpu.sync_copy(rows_v, out_hbm.at[pl.ds(base, _BPW)])

    return gather(emb_a, emb_v, idx_flat)


def kernel(z_e_x, emb_a, emb_v):
    emb_flat = jnp.stack([emb_a, emb_v]).reshape(2 * K, D)
    logits, idx = _logits_and_idx(z_e_x, emb_flat)
    idx_flat = idx.reshape(2 * B)
    rows = _sc_gather(emb_a, emb_v, idx_flat)
    z_q = jnp.concatenate([rows[:B], rows[B:]], axis=-1)
    return z_q, z_q, logits


# speedup vs baseline: 1.6803x; 1.6803x over previous
import functools

import jax
import jax.numpy as jnp
from jax import lax
from jax.experimental import pallas as pl
from jax.experimental.pallas import tpu as pltpu
from jax.experimental.pallas import tpu_sc as plsc

K = 8192
D = 256
B = 4096

BB = 512
BK = 2048
NB = B // BB
NK = K // BK
NS = NK * NB

_NC = 2
_NS = 16
_NW = _NC * _NS
_BPW = B // _NW


def _mm_argmax(z_ref, e_ref, out_ref, idx_ref, buf_ref, max_ref, arg_ref):
    s = pl.program_id(1)

    @pl.when(s > 0)
    def _():
        blk = buf_ref[...]
        sp = s - 1
        bkp = sp // NB
        bbp = lax.rem(sp, NB)
        m = jnp.max(blk, axis=1, keepdims=True)
        cols = lax.broadcasted_iota(jnp.int32, (BB, BK), 1)
        inblk = jnp.min(jnp.where(blk == m, cols, BK), axis=1, keepdims=True)
        gidx = bkp * BK + inblk
        rows = pl.ds(bbp * BB, BB)

        @pl.when(bkp == 0)
        def _():
            max_ref[rows, :] = m
            arg_ref[rows, :] = gidx

        @pl.when(bkp > 0)
        def _():
            cur = max_ref[rows, :]
            better = m > cur
            max_ref[rows, :] = jnp.where(better, m, cur)
            arg_ref[rows, :] = jnp.where(better, gidx, arg_ref[rows, :])

        @pl.when(bkp == NK - 1)
        def _():
            idx_ref[0] = arg_ref[rows, :]

    @pl.when(s < NS)
    def _():
        blk = lax.dot_general(z_ref[...], e_ref[...], (((1,), (1,)), ((), ())),
                              preferred_element_type=jnp.float32)
        out_ref[...] = blk
        buf_ref[...] = blk


def _logits_and_idx(z_e_x, emb_flat):
    def zmap(c, s):
        sp = jnp.minimum(s, NS - 1)
        return (lax.rem(sp, NB), c)

    def emap(c, s):
        sp = jnp.minimum(s, NS - 1)
        return (c * NK + sp // NB, 0)

    def omap(c, s):
        sp = jnp.minimum(s, NS - 1)
        return (lax.rem(sp, NB), c * NK + sp // NB)

    def imap(c, s):
        sp = jnp.maximum(s, 1) - 1
        return (c, lax.rem(sp, NB), 0)

    return pl.pallas_call(
        _mm_argmax,
        grid=(2, NS + 1),
        in_specs=[
            pl.BlockSpec((BB, D), zmap),
            pl.BlockSpec((BK, D), emap),
        ],
        out_specs=[
            pl.BlockSpec((BB, BK), omap),
            pl.BlockSpec((1, BB, 1), imap),
        ],
        out_shape=[
            jax.ShapeDtypeStruct((B, 2 * K), jnp.float32),
            jax.ShapeDtypeStruct((2, B, 1), jnp.int32),
        ],
        scratch_shapes=[
            pltpu.VMEM((BB, BK), jnp.float32),
            pltpu.VMEM((B, 1), jnp.float32),
            pltpu.VMEM((B, 1), jnp.int32),
        ],
        compiler_params=pltpu.CompilerParams(
            dimension_semantics=("arbitrary", "arbitrary")),
    )(z_e_x, emb_flat)


def _sc_gather(emb_a, emb_v, idx_flat):
    mesh = plsc.VectorSubcoreMesh(core_axis_name="c", subcore_axis_name="s")

    @functools.partial(
        pl.kernel, mesh=mesh,
        out_type=jax.ShapeDtypeStruct((2 * B, D), jnp.float32),
        scratch_types=[
            pltpu.VMEM((_BPW,), jnp.int32),
            pltpu.VMEM((_BPW, D), jnp.float32),
            pltpu.SemaphoreType.DMA,
        ],
    )
    def gather(ea_hbm, ev_hbm, idx_hbm, out_hbm, idx_v, rows_v, sem):
        wid = lax.axis_index("s") * _NC + lax.axis_index("c")
        for t, tab in ((0, ea_hbm), (1, ev_hbm)):
            base = t * B + wid * _BPW
            pltpu.sync_copy(idx_hbm.at[pl.ds(base, _BPW)], idx_v)
            pltpu.async_copy(tab.at[idx_v], rows_v, sem).wait()
            pltpu.sync_copy(rows_v, out_hbm.at[pl.ds(base, _BPW)])

    return gather(emb_a, emb_v, idx_flat)


def kernel(z_e_x, emb_a, emb_v):
    emb_flat = jnp.stack([emb_a, emb_v]).reshape(2 * K, D)
    logits, idx = _logits_and_idx(z_e_x, emb_flat)
    idx_flat = idx.reshape(2 * B)
    rows = _sc_gather(emb_a, emb_v, idx_flat)
    z_q = jnp.concatenate([rows[:B], rows[B:]], axis=-1)
    return z_q, z_q, logits


# speedup vs baseline: 1.8902x; 1.1249x over previous
import functools

import jax
import jax.numpy as jnp
from jax import lax
from jax.experimental import pallas as pl
from jax.experimental.pallas import tpu as pltpu
from jax.experimental.pallas import tpu_sc as plsc

K = 8192
D = 256
B = 4096

BB = 1024
BK = 2048
NB = B // BB
NK = K // BK
NS = NK * NB

_NC = 2
_NS = 16
_NW = _NC * _NS
_BPW = B // _NW


def _mm_argmax(z_ref, e_ref, out_ref, idx_ref, buf_ref, max_ref, arg_ref):
    s = pl.program_id(1)

    @pl.when(s > 0)
    def _():
        blk = buf_ref[...]
        sp = s - 1
        bkp = sp // NB
        bbp = lax.rem(sp, NB)
        m = jnp.max(blk, axis=1, keepdims=True)
        cols = lax.broadcasted_iota(jnp.int32, (BB, BK), 1)
        inblk = jnp.min(jnp.where(blk == m, cols, BK), axis=1, keepdims=True)
        gidx = bkp * BK + inblk
        rows = pl.ds(bbp * BB, BB)

        @pl.when(bkp == 0)
        def _():
            max_ref[rows, :] = m
            arg_ref[rows, :] = gidx

        @pl.when(bkp > 0)
        def _():
            cur = max_ref[rows, :]
            better = m > cur
            max_ref[rows, :] = jnp.where(better, m, cur)
            arg_ref[rows, :] = jnp.where(better, gidx, arg_ref[rows, :])

        @pl.when(bkp == NK - 1)
        def _():
            idx_ref[0] = arg_ref[rows, :]

    @pl.when(s < NS)
    def _():
        blk = lax.dot_general(z_ref[...], e_ref[...], (((1,), (1,)), ((), ())),
                              preferred_element_type=jnp.float32)
        out_ref[...] = blk
        buf_ref[...] = blk


def _logits_and_idx(z_e_x, emb_flat):
    def zmap(c, s):
        sp = jnp.minimum(s, NS - 1)
        return (lax.rem(sp, NB), c)

    def emap(c, s):
        sp = jnp.minimum(s, NS - 1)
        return (c * NK + sp // NB, 0)

    def omap(c, s):
        sp = jnp.minimum(s, NS - 1)
        return (lax.rem(sp, NB), c * NK + sp // NB)

    def imap(c, s):
        sp = jnp.maximum(s, 1) - 1
        return (c, lax.rem(sp, NB), 0)

    return pl.pallas_call(
        _mm_argmax,
        grid=(2, NS + 1),
        in_specs=[
            pl.BlockSpec((BB, D), zmap),
            pl.BlockSpec((BK, D), emap),
        ],
        out_specs=[
            pl.BlockSpec((BB, BK), omap),
            pl.BlockSpec((1, BB, 1), imap),
        ],
        out_shape=[
            jax.ShapeDtypeStruct((B, 2 * K), jnp.float32),
            jax.ShapeDtypeStruct((2, B, 1), jnp.int32),
        ],
        scratch_shapes=[
            pltpu.VMEM((BB, BK), jnp.float32),
            pltpu.VMEM((B, 1), jnp.float32),
            pltpu.VMEM((B, 1), jnp.int32),
        ],
        compiler_params=pltpu.CompilerParams(
            dimension_semantics=("arbitrary", "arbitrary")),
    )(z_e_x, emb_flat)


def _sc_gather(emb_a, emb_v, idx_flat):
    mesh = plsc.VectorSubcoreMesh(core_axis_name="c", subcore_axis_name="s")

    @functools.partial(
        pl.kernel, mesh=mesh,
        out_type=jax.ShapeDtypeStruct((2 * B, D), jnp.float32),
        scratch_types=[
            pltpu.VMEM((_BPW,), jnp.int32),
            pltpu.VMEM((_BPW, D), jnp.float32),
            pltpu.SemaphoreType.DMA,
        ],
    )
    def gather(ea_hbm, ev_hbm, idx_hbm, out_hbm, idx_v, rows_v, sem):
        wid = lax.axis_index("s") * _NC + lax.axis_index("c")
        for t, tab in ((0, ea_hbm), (1, ev_hbm)):
            base = t * B + wid * _BPW
            pltpu.sync_copy(idx_hbm.at[pl.ds(base, _BPW)], idx_v)
            pltpu.async_copy(tab.at[idx_v], rows_v, sem).wait()
            pltpu.sync_copy(rows_v, out_hbm.at[pl.ds(base, _BPW)])

    return gather(emb_a, emb_v, idx_flat)


def kernel(z_e_x, emb_a, emb_v):
    emb_flat = jnp.stack([emb_a, emb_v]).reshape(2 * K, D)
    logits, idx = _logits_and_idx(z_e_x, emb_flat)
    idx_flat = idx.reshape(2 * B)
    rows = _sc_gather(emb_a, emb_v, idx_flat)
    z_q = jnp.concatenate([rows[:B], rows[B:]], axis=-1)
    return z_q, z_q, logits


# speedup vs baseline: 1.9607x; 1.0373x over previous
import functools

import jax
import jax.numpy as jnp
from jax import lax
from jax.experimental import pallas as pl
from jax.experimental.pallas import tpu as pltpu
from jax.experimental.pallas import tpu_sc as plsc

K = 8192
D = 256
B = 4096

BB = 1024
BK = 2048
NB = B // BB
NK = K // BK

_NC = 2
_NS = 16
_NW = _NC * _NS
_BPW = B // _NW


def _mm_argmax(z_ref, e_ref, out_ref, idx_ref, max_ref, arg_ref):
    bk = pl.program_id(1)
    bb = pl.program_id(2)
    blk = lax.dot_general(z_ref[...], e_ref[...], (((1,), (1,)), ((), ())),
                          preferred_element_type=jnp.float32)
    out_ref[...] = blk

    m = jnp.max(blk, axis=1, keepdims=True)
    cols = lax.broadcasted_iota(jnp.int32, (BB, BK), 1)
    inblk = jnp.min(jnp.where(blk == m, cols, BK), axis=1, keepdims=True)
    gidx = bk * BK + inblk
    rows = pl.ds(bb * BB, BB)

    @pl.when(bk == 0)
    def _():
        max_ref[rows, :] = m
        arg_ref[rows, :] = gidx

    @pl.when(bk > 0)
    def _():
        cur = max_ref[rows, :]
        better = m > cur
        max_ref[rows, :] = jnp.where(better, m, cur)
        arg_ref[rows, :] = jnp.where(better, gidx, arg_ref[rows, :])

    @pl.when(bk == NK - 1)
    def _():
        idx_ref[0] = arg_ref[rows, :]


def _logits_and_idx(z_e_x, emb_flat):
    return pl.pallas_call(
        _mm_argmax,
        grid=(2, NK, NB),
        in_specs=[
            pl.BlockSpec((BB, D), lambda c, k, b: (b, c)),
            pl.BlockSpec((BK, D), lambda c, k, b: (c * NK + k, 0)),
        ],
        out_specs=[
            pl.BlockSpec((BB, BK), lambda c, k, b: (b, c * NK + k)),
            pl.BlockSpec((1, BB, 1), lambda c, k, b: (c, b, 0)),
        ],
        out_shape=[
            jax.ShapeDtypeStruct((B, 2 * K), jnp.float32),
            jax.ShapeDtypeStruct((2, B, 1), jnp.int32),
        ],
        scratch_shapes=[
            pltpu.VMEM((B, 1), jnp.float32),
            pltpu.VMEM((B, 1), jnp.int32),
        ],
        compiler_params=pltpu.CompilerParams(
            dimension_semantics=("arbitrary", "arbitrary", "arbitrary")),
    )(z_e_x, emb_flat)


def _sc_gather(emb_a, emb_v, idx_flat):
    mesh = plsc.VectorSubcoreMesh(core_axis_name="c", subcore_axis_name="s")

    @functools.partial(
        pl.kernel, mesh=mesh,
        out_type=jax.ShapeDtypeStruct((2 * B, D), jnp.float32),
        scratch_types=[
            pltpu.VMEM((_BPW,), jnp.int32),
            pltpu.VMEM((_BPW, D), jnp.float32),
            pltpu.SemaphoreType.DMA,
        ],
    )
    def gather(ea_hbm, ev_hbm, idx_hbm, out_hbm, idx_v, rows_v, sem):
        wid = lax.axis_index("s") * _NC + lax.axis_index("c")
        for t, tab in ((0, ea_hbm), (1, ev_hbm)):
            base = t * B + wid * _BPW
            pltpu.sync_copy(idx_hbm.at[pl.ds(base, _BPW)], idx_v)
            pltpu.async_copy(tab.at[idx_v], rows_v, sem).wait()
            pltpu.sync_copy(rows_v, out_hbm.at[pl.ds(base, _BPW)])

    return gather(emb_a, emb_v, idx_flat)


def kernel(z_e_x, emb_a, emb_v):
    emb_flat = jnp.stack([emb_a, emb_v]).reshape(2 * K, D)
    logits, idx = _logits_and_idx(z_e_x, emb_flat)
    idx_flat = idx.reshape(2 * B)
    rows = _sc_gather(emb_a, emb_v, idx_flat)
    z_q = jnp.concatenate([rows[:B], rows[B:]], axis=-1)
    return z_q, z_q, logits


# speedup vs baseline: 2.0470x; 1.0440x over previous
import functools

import jax
import jax.numpy as jnp
from jax import lax
from jax.experimental import pallas as pl
from jax.experimental.pallas import tpu as pltpu
from jax.experimental.pallas import tpu_sc as plsc

K = 8192
D = 256
B = 4096

BB = 2048
BK = 2048
NB = B // BB
NK = K // BK

_NC = 2
_NS = 16
_NW = _NC * _NS
_BPW = B // _NW


def _mm_argmax(z_ref, e_ref, out_ref, idx_ref, max_ref, arg_ref):
    bk = pl.program_id(1)
    bb = pl.program_id(2)
    blk = lax.dot_general(z_ref[...], e_ref[...], (((1,), (1,)), ((), ())),
                          preferred_element_type=jnp.float32)
    out_ref[...] = blk

    m = jnp.max(blk, axis=1, keepdims=True)
    cols = lax.broadcasted_iota(jnp.int32, (BB, BK), 1)
    inblk = jnp.min(jnp.where(blk == m, cols, BK), axis=1, keepdims=True)
    gidx = bk * BK + inblk
    rows = pl.ds(bb * BB, BB)

    @pl.when(bk == 0)
    def _():
        max_ref[rows, :] = m
        arg_ref[rows, :] = gidx

    @pl.when(bk > 0)
    def _():
        cur = max_ref[rows, :]
        better = m > cur
        max_ref[rows, :] = jnp.where(better, m, cur)
        arg_ref[rows, :] = jnp.where(better, gidx, arg_ref[rows, :])

    @pl.when(bk == NK - 1)
    def _():
        idx_ref[0] = arg_ref[rows, :]


def _logits_and_idx(z_e_x, emb_flat):
    return pl.pallas_call(
        _mm_argmax,
        grid=(2, NK, NB),
        in_specs=[
            pl.BlockSpec((BB, D), lambda c, k, b: (b, c)),
            pl.BlockSpec((BK, D), lambda c, k, b: (c * NK + k, 0)),
        ],
        out_specs=[
            pl.BlockSpec((BB, BK), lambda c, k, b: (b, c * NK + k)),
            pl.BlockSpec((1, BB, 1), lambda c, k, b: (c, b, 0)),
        ],
        out_shape=[
            jax.ShapeDtypeStruct((B, 2 * K), jnp.float32),
            jax.ShapeDtypeStruct((2, B, 1), jnp.int32),
        ],
        scratch_shapes=[
            pltpu.VMEM((B, 1), jnp.float32),
            pltpu.VMEM((B, 1), jnp.int32),
        ],
        compiler_params=pltpu.CompilerParams(
            dimension_semantics=("arbitrary", "arbitrary", "arbitrary")),
    )(z_e_x, emb_flat)


def _sc_gather(emb_a, emb_v, idx_flat):
    mesh = plsc.VectorSubcoreMesh(core_axis_name="c", subcore_axis_name="s")

    @functools.partial(
        pl.kernel, mesh=mesh,
        out_type=jax.ShapeDtypeStruct((2 * B, D), jnp.float32),
        scratch_types=[
            pltpu.VMEM((_BPW,), jnp.int32),
            pltpu.VMEM((_BPW, D), jnp.float32),
            pltpu.SemaphoreType.DMA,
        ],
    )
    def gather(ea_hbm, ev_hbm, idx_hbm, out_hbm, idx_v, rows_v, sem):
        wid = lax.axis_index("s") * _NC + lax.axis_index("c")
        for t, tab in ((0, ea_hbm), (1, ev_hbm)):
            base = t * B + wid * _BPW
            pltpu.sync_copy(idx_hbm.at[pl.ds(base, _BPW)], idx_v)
            pltpu.async_copy(tab.at[idx_v], rows_v, sem).wait()
            pltpu.sync_copy(rows_v, out_hbm.at[pl.ds(base, _BPW)])

    return gather(emb_a, emb_v, idx_flat)


def kernel(z_e_x, emb_a, emb_v):
    emb_flat = jnp.stack([emb_a, emb_v]).reshape(2 * K, D)
    logits, idx = _logits_and_idx(z_e_x, emb_flat)
    idx_flat = idx.reshape(2 * B)
    rows = _sc_gather(emb_a, emb_v, idx_flat)
    z_q = jnp.concatenate([rows[:B], rows[B:]], axis=-1)
    return z_q, z_q, logits


# speedup vs baseline: 2.0875x; 1.0198x over previous
import functools

import jax
import jax.numpy as jnp
from jax import lax
from jax.experimental import pallas as pl
from jax.experimental.pallas import tpu as pltpu
from jax.experimental.pallas import tpu_sc as plsc

K = 8192
D = 256
B = 4096

BB = 4096
BK = 1024
NB = B // BB
NK = K // BK

_NC = 2
_NS = 16
_NW = _NC * _NS
_BPW = B // _NW


def _mm_argmax(z_ref, e_ref, out_ref, idx_ref, max_ref, arg_ref):
    bk = pl.program_id(1)
    bb = pl.program_id(2)
    blk = lax.dot_general(z_ref[...], e_ref[...], (((1,), (1,)), ((), ())),
                          preferred_element_type=jnp.float32)
    out_ref[...] = blk

    m = jnp.max(blk, axis=1, keepdims=True)
    cols = lax.broadcasted_iota(jnp.int32, (BB, BK), 1)
    inblk = jnp.min(jnp.where(blk == m, cols, BK), axis=1, keepdims=True)
    gidx = bk * BK + inblk
    rows = pl.ds(bb * BB, BB)

    @pl.when(bk == 0)
    def _():
        max_ref[rows, :] = m
        arg_ref[rows, :] = gidx

    @pl.when(bk > 0)
    def _():
        cur = max_ref[rows, :]
        better = m > cur
        max_ref[rows, :] = jnp.where(better, m, cur)
        arg_ref[rows, :] = jnp.where(better, gidx, arg_ref[rows, :])

    @pl.when(bk == NK - 1)
    def _():
        idx_ref[0] = arg_ref[rows, :]


def _logits_and_idx(z_e_x, emb_flat):
    return pl.pallas_call(
        _mm_argmax,
        grid=(2, NK, NB),
        in_specs=[
            pl.BlockSpec((BB, D), lambda c, k, b: (b, c)),
            pl.BlockSpec((BK, D), lambda c, k, b: (c * NK + k, 0)),
        ],
        out_specs=[
            pl.BlockSpec((BB, BK), lambda c, k, b: (b, c * NK + k)),
            pl.BlockSpec((1, BB, 1), lambda c, k, b: (c, b, 0)),
        ],
        out_shape=[
            jax.ShapeDtypeStruct((B, 2 * K), jnp.float32),
            jax.ShapeDtypeStruct((2, B, 1), jnp.int32),
        ],
        scratch_shapes=[
            pltpu.VMEM((B, 1), jnp.float32),
            pltpu.VMEM((B, 1), jnp.int32),
        ],
        compiler_params=pltpu.CompilerParams(
            dimension_semantics=("arbitrary", "arbitrary", "arbitrary")),
    )(z_e_x, emb_flat)


def _sc_gather(emb_a, emb_v, idx_flat):
    mesh = plsc.VectorSubcoreMesh(core_axis_name="c", subcore_axis_name="s")

    @functools.partial(
        pl.kernel, mesh=mesh,
        out_type=jax.ShapeDtypeStruct((2 * B, D), jnp.float32),
        scratch_types=[
            pltpu.VMEM((_BPW,), jnp.int32),
            pltpu.VMEM((_BPW, D), jnp.float32),
            pltpu.SemaphoreType.DMA,
        ],
    )
    def gather(ea_hbm, ev_hbm, idx_hbm, out_hbm, idx_v, rows_v, sem):
        wid = lax.axis_index("s") * _NC + lax.axis_index("c")
        for t, tab in ((0, ea_hbm), (1, ev_hbm)):
            base = t * B + wid * _BPW
            pltpu.sync_copy(idx_hbm.at[pl.ds(base, _BPW)], idx_v)
            pltpu.async_copy(tab.at[idx_v], rows_v, sem).wait()
            pltpu.sync_copy(rows_v, out_hbm.at[pl.ds(base, _BPW)])

    return gather(emb_a, emb_v, idx_flat)


def kernel(z_e_x, emb_a, emb_v):
    emb_flat = jnp.stack([emb_a, emb_v]).reshape(2 * K, D)
    logits, idx = _logits_and_idx(z_e_x, emb_flat)
    idx_flat = idx.reshape(2 * B)
    rows = _sc_gather(emb_a, emb_v, idx_flat)
    z_q = jnp.concatenate([rows[:B], rows[B:]], axis=-1)
    return z_q, z_q, logits
